# baseline (speedup 1.0000x reference)
import functools

import jax
import jax.numpy as jnp
from jax import lax
from jax.experimental import pallas as pl
from jax.experimental.pallas import tpu as pltpu
from jax.experimental.pallas import tpu_sc as plsc

_NBUF = 4
_CHUNK = 40


@functools.cache
def _make_lookup(B, V, D, NC, NS):
    NW = NC * NS
    b_per_w = B // NW
    C = _CHUNK
    NB = _NBUF
    n_chunks = b_per_w // C
    n_groups = n_chunks // NB
    mesh = plsc.VectorSubcoreMesh(core_axis_name="c", subcore_axis_name="s")

    scratch = (
        [pltpu.VMEM((C,), jnp.int32) for _ in range(NB)]
        + [pltpu.VMEM((C, D), jnp.float32) for _ in range(NB)]
        + [pltpu.SemaphoreType.DMA for _ in range(3 * NB)]
    )

    @functools.partial(
        pl.kernel,
        out_type=jax.ShapeDtypeStruct((B, D), jnp.float32),
        mesh=mesh,
        scratch_types=scratch,
    )
    def lookup(idx_hbm, table_hbm, out_hbm, *scr):
        ibufs = scr[:NB]
        bufs = scr[NB:2 * NB]
        isems = scr[2 * NB:3 * NB]
        gsems = scr[3 * NB:4 * NB]
        ssems = scr[4 * NB:5 * NB]

        wid = lax.axis_index("s") * NC + lax.axis_index("c")
        base = wid * b_per_w

        def idx_load(i, b):
            pltpu.async_copy(
                idx_hbm.at[pl.ds(base + i * C, C)], ibufs[b], isems[b])

        def wait_idx(b):
            pltpu.make_async_copy(
                idx_hbm.at[pl.ds(base, C)], ibufs[b], isems[b]).wait()

        def gather(b):
            pltpu.async_copy(table_hbm.at[ibufs[b]], bufs[b], gsems[b])

        def wait_gather(b):
            pltpu.make_async_copy(
                table_hbm.at[ibufs[b]], bufs[b], gsems[b]).wait()

        def store(i, b):
            pltpu.async_copy(
                bufs[b], out_hbm.at[pl.ds(base + i * C, C)], ssems[b])

        def wait_store(b):
            pltpu.make_async_copy(
                bufs[b], out_hbm.at[pl.ds(base, C)], ssems[b]).wait()

        for b in range(NB):
            idx_load(b, b)
        wait_idx(0)
        gather(0)

        def body(j, carry):
            for b in range(NB):
                i = j * NB + b
                nb = (b + 1) % NB

                @pl.when(i >= NB - 1)
                def _():
                    wait_store(nb)

                @pl.when(i + 1 < n_chunks)
                def _():
                    wait_idx(nb)
                    gather(nb)

                wait_gather(b)

                @pl.when(i + NB < n_chunks)
                def _():
                    idx_load(i + NB, b)

                store(i, b)
            return carry

        lax.fori_loop(0, n_groups, body, 0)
        for k in range(1, NB):
            wait_store((n_chunks - NB + k) % NB)

    return lookup


def kernel(x, table):
    Bt, L = x.shape
    V, D = table.shape
    B = Bt * L
    info = plsc.get_sparse_core_info()
    lookup = _make_lookup(B, V, D, info.num_cores, info.num_subcores)
    out = lookup(x.reshape(B), table)
    return out.reshape(Bt, L, D)


# speedup vs baseline: 1.0064x; 1.0064x over previous
import functools

import jax
import jax.numpy as jnp
from jax import lax
from jax.experimental import pallas as pl
from jax.experimental.pallas import tpu as pltpu
from jax.experimental.pallas import tpu_sc as plsc

_NBUF = 4
_CHUNK = 40


@functools.cache
def _make_lookup(B, V, D, NC, NS):
    NW = NC * NS
    b_per_w = B // NW
    C = _CHUNK
    NB = _NBUF
    n_chunks = b_per_w // C
    n_groups = n_chunks // NB
    mesh = plsc.VectorSubcoreMesh(core_axis_name="c", subcore_axis_name="s")

    scratch = (
        [pltpu.VMEM((C,), jnp.int32) for _ in range(NB)]
        + [pltpu.VMEM((C, D), jnp.float32) for _ in range(NB)]
        + [pltpu.SemaphoreType.DMA for _ in range(3 * NB)]
    )

    @functools.partial(
        pl.kernel,
        out_type=jax.ShapeDtypeStruct((B, D), jnp.float32),
        mesh=mesh,
        scratch_types=scratch,
    )
    def lookup(idx_hbm, table_hbm, out_hbm, *scr):
        ibufs = scr[:NB]
        bufs = scr[NB:2 * NB]
        isems = scr[2 * NB:3 * NB]
        gsems = scr[3 * NB:4 * NB]
        ssems = scr[4 * NB:5 * NB]

        wid = lax.axis_index("s") * NC + lax.axis_index("c")

        def chunk_off(i):
            return (wid + i * NW) * C

        def idx_load(i, b):
            pltpu.async_copy(
                idx_hbm.at[pl.ds(chunk_off(i), C)], ibufs[b], isems[b])

        def wait_idx(b):
            pltpu.make_async_copy(
                idx_hbm.at[pl.ds(0, C)], ibufs[b], isems[b]).wait()

        def gather(b):
            pltpu.async_copy(table_hbm.at[ibufs[b]], bufs[b], gsems[b])

        def wait_gather(b):
            pltpu.make_async_copy(
                table_hbm.at[ibufs[b]], bufs[b], gsems[b]).wait()

        def store(i, b):
            pltpu.async_copy(
                bufs[b], out_hbm.at[pl.ds(chunk_off(i), C)], ssems[b])

        def wait_store(b):
            pltpu.make_async_copy(
                bufs[b], out_hbm.at[pl.ds(0, C)], ssems[b]).wait()

        for b in range(NB):
            idx_load(b, b)
        wait_idx(0)
        gather(0)

        def body(j, carry):
            for b in range(NB):
                i = j * NB + b
                nb = (b + 1) % NB

                @pl.when(i >= NB - 1)
                def _():
                    wait_store(nb)

                @pl.when(i + 1 < n_chunks)
                def _():
                    wait_idx(nb)
                    gather(nb)

                wait_gather(b)

                @pl.when(i + NB < n_chunks)
                def _():
                    idx_load(i + NB, b)

                store(i, b)
            return carry

        lax.fori_loop(0, n_groups, body, 0)
        for k in range(1, NB):
            wait_store((n_chunks - NB + k) % NB)

    return lookup


def kernel(x, table):
    Bt, L = x.shape
    V, D = table.shape
    B = Bt * L
    info = plsc.get_sparse_core_info()
    lookup = _make_lookup(B, V, D, info.num_cores, info.num_subcores)
    out = lookup(x.reshape(B), table)
    return out.reshape(Bt, L, D)
